# Initial kernel scaffold; baseline (speedup 1.0000x reference)
import jax
import jax.numpy as jnp
from jax.experimental import pallas as pl


def kernel(left_idx, up_idx, table):
    raise NotImplementedError("write your pallas kernel here")


# speedup vs baseline: 1.0618x; 1.0618x over previous
import functools

import jax
import jax.numpy as jnp
from jax import lax
from jax.experimental import pallas as pl
from jax.experimental.pallas import tpu as pltpu
from jax.experimental.pallas import tpu_sc as plsc

D = 384
N_TOK = 4096 * 200
NC, NS = 2, 16
NW = NC * NS
T_PER_W = N_TOK // NW
CH = 128
N_CH = T_PER_W // CH


def _sc_gather_concat(left_flat, up_flat, table):
    mesh = plsc.VectorSubcoreMesh(core_axis_name="c", subcore_axis_name="s")

    @functools.partial(
        pl.kernel,
        out_type=jax.ShapeDtypeStruct((N_TOK, 2, D), jnp.float32),
        mesh=mesh,
        scratch_types=[
            pltpu.VMEM((CH,), jnp.int32),
            pltpu.VMEM((CH,), jnp.int32),
            pltpu.VMEM((CH, D), jnp.float32),
            pltpu.VMEM((CH, D), jnp.float32),
            pltpu.SemaphoreType.DMA,
            pltpu.SemaphoreType.DMA,
        ],
    )
    def k(left_hbm, up_hbm, table_hbm, out_hbm,
          idx_l, idx_u, rows_l, rows_u, sem_l, sem_u):
        wid = lax.axis_index("s") * NC + lax.axis_index("c")
        base = wid * T_PER_W

        def body(c, carry):
            tok = base + c * CH
            pltpu.sync_copy(left_hbm.at[pl.ds(tok, CH)], idx_l)
            pltpu.sync_copy(up_hbm.at[pl.ds(tok, CH)], idx_u)
            cp_l = pltpu.async_copy(table_hbm.at[idx_l], rows_l, sem_l)
            cp_u = pltpu.async_copy(table_hbm.at[idx_u], rows_u, sem_u)
            cp_l.wait()
            cp_u.wait()
            pltpu.sync_copy(rows_l, out_hbm.at[pl.ds(tok, CH), 0])
            pltpu.sync_copy(rows_u, out_hbm.at[pl.ds(tok, CH), 1])
            return carry

        lax.fori_loop(0, N_CH, body, 0)

    return k(left_flat, up_flat, table)


def kernel(left_idx, up_idx, table):
    B, L = left_idx.shape
    out = _sc_gather_concat(left_idx.reshape(-1), up_idx.reshape(-1), table)
    return out.reshape(B, L, 2 * D)


# speedup vs baseline: 1.8309x; 1.7244x over previous
import functools

import jax
import jax.numpy as jnp
from jax import lax
from jax.experimental import pallas as pl
from jax.experimental.pallas import tpu as pltpu
from jax.experimental.pallas import tpu_sc as plsc

D = 384
N_TOK = 4096 * 200
N_JOB = 2 * N_TOK
NC, NS = 2, 16
NW = NC * NS
T_PER_W = N_JOB // NW
J = 80
N_CH = T_PER_W // J


def _sc_gather_concat(idx_pairs, table):
    mesh = plsc.VectorSubcoreMesh(core_axis_name="c", subcore_axis_name="s")

    @functools.partial(
        pl.kernel,
        out_type=jax.ShapeDtypeStruct((N_JOB, D), jnp.float32),
        mesh=mesh,
        scratch_types=[
            pltpu.VMEM((T_PER_W,), jnp.int32),
            pltpu.VMEM((J, D), jnp.float32),
            pltpu.VMEM((J, D), jnp.float32),
            pltpu.SemaphoreType.DMA,
            pltpu.SemaphoreType.DMA,
            pltpu.SemaphoreType.DMA,
            pltpu.SemaphoreType.DMA,
        ],
    )
    def k(idx_hbm, table_hbm, out_hbm,
          idx_all, rows0, rows1, sem_g0, sem_g1, sem_w0, sem_w1):
        wid = lax.axis_index("s") * NC + lax.axis_index("c")
        base = wid * T_PER_W

        rows = (rows0, rows1)
        sem_g = (sem_g0, sem_g1)
        sem_w = (sem_w0, sem_w1)

        def start_gather(c, p):
            pltpu.async_copy(
                table_hbm.at[idx_all.at[pl.ds(c * J, J)]], rows[p], sem_g[p])

        def wait_gather(p):
            pltpu.make_async_copy(out_hbm.at[pl.ds(0, J)], rows[p],
                                  sem_g[p]).wait()

        def start_write(c, p):
            pltpu.async_copy(rows[p], out_hbm.at[pl.ds(base + c * J, J)],
                             sem_w[p])

        def wait_write(p):
            pltpu.make_async_copy(rows[p], out_hbm.at[pl.ds(0, J)],
                                  sem_w[p]).wait()

        pltpu.sync_copy(idx_hbm.at[pl.ds(base, T_PER_W)], idx_all)
        start_gather(0, 0)

        def pair_body(i, carry):
            c0 = 2 * i
            wait_gather(0)
            start_write(c0, 0)

            @pl.when(i > 0)
            def _():
                wait_write(1)
            start_gather(c0 + 1, 1)

            wait_gather(1)
            start_write(c0 + 1, 1)

            @pl.when(i < N_CH // 2 - 1)
            def _():
                wait_write(0)
                start_gather(c0 + 2, 0)
            return carry

        lax.fori_loop(0, N_CH // 2, pair_body, 0)
        wait_write(0)
        wait_write(1)

    return k(idx_pairs, table)


def kernel(left_idx, up_idx, table):
    B, L = left_idx.shape
    idx_pairs = jnp.stack(
        (left_idx.reshape(-1), up_idx.reshape(-1)), axis=-1).reshape(-1)
    out = _sc_gather_concat(idx_pairs, table)
    return out.reshape(B, L, 2 * D)


# speedup vs baseline: 1.8390x; 1.0044x over previous
import functools

import jax
import jax.numpy as jnp
from jax import lax
from jax.experimental import pallas as pl
from jax.experimental.pallas import tpu as pltpu
from jax.experimental.pallas import tpu_sc as plsc

D = 384
N_TOK = 4096 * 200
N_JOB = 2 * N_TOK
NC, NS = 2, 16
NW = NC * NS
T_PER_W = N_JOB // NW
J = 40
N_CH = T_PER_W // J
NBUF = 4
N_IT = N_CH // NBUF


def _sc_gather_concat(idx_pairs, table):
    mesh = plsc.VectorSubcoreMesh(core_axis_name="c", subcore_axis_name="s")

    @functools.partial(
        pl.kernel,
        out_type=jax.ShapeDtypeStruct((N_JOB, D), jnp.float32),
        mesh=mesh,
        scratch_types=(
            [pltpu.VMEM((T_PER_W,), jnp.int32)]
            + [pltpu.VMEM((J, D), jnp.float32) for _ in range(NBUF)]
            + [pltpu.SemaphoreType.DMA for _ in range(2 * NBUF)]
        ),
    )
    def k(idx_hbm, table_hbm, out_hbm, idx_all, *bufs):
        rows = bufs[:NBUF]
        sem_g = bufs[NBUF:2 * NBUF]
        sem_w = bufs[2 * NBUF:]
        wid = lax.axis_index("s") * NC + lax.axis_index("c")
        base = wid * T_PER_W

        def start_gather(c, p):
            pltpu.async_copy(
                table_hbm.at[idx_all.at[pl.ds(c * J, J)]], rows[p], sem_g[p])

        def wait_gather(p):
            pltpu.make_async_copy(out_hbm.at[pl.ds(0, J)], rows[p],
                                  sem_g[p]).wait()

        def start_write(c, p):
            pltpu.async_copy(rows[p], out_hbm.at[pl.ds(base + c * J, J)],
                             sem_w[p])

        def wait_write(p):
            pltpu.make_async_copy(rows[p], out_hbm.at[pl.ds(0, J)],
                                  sem_w[p]).wait()

        pltpu.sync_copy(idx_hbm.at[pl.ds(base, T_PER_W)], idx_all)
        for p in range(NBUF - 1):
            start_gather(p, p)

        def body(i, carry):
            c0 = NBUF * i
            for b in range(NBUF):
                q = (b - 1) % NBUF
                wait_gather(b)
                start_write(c0 + b, b)
                if b == 0:
                    @pl.when(i > 0)
                    def _():
                        wait_write(q)
                    start_gather(c0 + NBUF - 1, q)
                else:
                    @pl.when(i < N_IT - 1)
                    def _():
                        wait_write(q)
                        start_gather(c0 + b + NBUF - 1, q)
            return carry

        lax.fori_loop(0, N_IT, body, 0)
        for p in range(NBUF):
            wait_write(p)

    return k(idx_pairs, table)


def kernel(left_idx, up_idx, table):
    B, L = left_idx.shape
    idx_pairs = jnp.stack(
        (left_idx.reshape(-1), up_idx.reshape(-1)), axis=-1).reshape(-1)
    out = _sc_gather_concat(idx_pairs, table)
    return out.reshape(B, L, 2 * D)


# speedup vs baseline: 1.8411x; 1.0011x over previous
import functools

import jax
import jax.numpy as jnp
from jax import lax
from jax.experimental import pallas as pl
from jax.experimental.pallas import tpu as pltpu
from jax.experimental.pallas import tpu_sc as plsc

D = 384
N_TOK = 4096 * 200
N_JOB = 2 * N_TOK
NC, NS = 2, 16
NW = NC * NS
T_PER_W = N_JOB // NW
J = 40
N_CH = T_PER_W // J
NBUF = 4
N_IT = N_CH // NBUF


def _sc_gather_concat(idx_pairs, table):
    mesh = plsc.VectorSubcoreMesh(core_axis_name="c", subcore_axis_name="s")

    @functools.partial(
        pl.kernel,
        out_type=jax.ShapeDtypeStruct((N_JOB, D), jnp.float32),
        mesh=mesh,
        compiler_params=pltpu.CompilerParams(use_tc_tiling_on_sc=True),
        scratch_types=(
            [pltpu.VMEM((T_PER_W,), jnp.int32)]
            + [pltpu.VMEM((J, D), jnp.float32) for _ in range(NBUF)]
            + [pltpu.SemaphoreType.DMA for _ in range(2 * NBUF)]
        ),
    )
    def k(idx_hbm, table_hbm, out_hbm, idx_all, *bufs):
        rows = bufs[:NBUF]
        sem_g = bufs[NBUF:2 * NBUF]
        sem_w = bufs[2 * NBUF:]
        wid = lax.axis_index("s") * NC + lax.axis_index("c")
        base = wid * T_PER_W

        def start_gather(c, p):
            pltpu.async_copy(
                table_hbm.at[idx_all.at[pl.ds(c * J, J)]], rows[p], sem_g[p])

        def wait_gather(p):
            pltpu.make_async_copy(out_hbm.at[pl.ds(0, J)], rows[p],
                                  sem_g[p]).wait()

        def start_write(c, p):
            pltpu.async_copy(rows[p], out_hbm.at[pl.ds(base + c * J, J)],
                             sem_w[p])

        def wait_write(p):
            pltpu.make_async_copy(rows[p], out_hbm.at[pl.ds(0, J)],
                                  sem_w[p]).wait()

        pltpu.sync_copy(idx_hbm.at[pl.ds(base, T_PER_W)], idx_all)
        for p in range(NBUF - 1):
            start_gather(p, p)

        def body(i, carry):
            c0 = NBUF * i
            for b in range(NBUF):
                q = (b - 1) % NBUF
                wait_gather(b)
                start_write(c0 + b, b)
                if b == 0:
                    @pl.when(i > 0)
                    def _():
                        wait_write(q)
                    start_gather(c0 + NBUF - 1, q)
                else:
                    @pl.when(i < N_IT - 1)
                    def _():
                        wait_write(q)
                        start_gather(c0 + b + NBUF - 1, q)
            return carry

        lax.fori_loop(0, N_IT, body, 0)
        for p in range(NBUF):
            wait_write(p)

    return k(idx_pairs, table)


def kernel(left_idx, up_idx, table):
    B, L = left_idx.shape
    idx_pairs = jnp.stack(
        (left_idx.reshape(-1), up_idx.reshape(-1)), axis=-1).reshape(-1)
    out = _sc_gather_concat(idx_pairs, table)
    return out.reshape(B, L, 2 * D)


# speedup vs baseline: 4.3768x; 2.3773x over previous
import functools

import jax
import jax.numpy as jnp
from jax import lax
from jax.experimental import pallas as pl
from jax.experimental.pallas import tpu as pltpu
from jax.experimental.pallas import tpu_sc as plsc

B, L = 4096, 200
D = 384
N_TOK = B * L
N_PIECE = 6 * N_TOK
NC, NS = 2, 16
NW = NC * NS
P_PER_W = N_PIECE // NW
JP = 240
N_CH = P_PER_W // JP
N_PAIR = N_CH // 2


def _sc_piece_gather(pidx, tp):
    mesh = plsc.VectorSubcoreMesh(core_axis_name="c", subcore_axis_name="s")

    @functools.partial(
        pl.kernel,
        out_type=jax.ShapeDtypeStruct((N_PIECE, 128), jnp.float32),
        mesh=mesh,
        scratch_types=(
            [pltpu.VMEM((2 * JP,), jnp.int32) for _ in range(4)]
            + [pltpu.VMEM((JP, 128), jnp.float32) for _ in range(2)]
            + [pltpu.SemaphoreType.DMA for _ in range(8)]
        ),
    )
    def k(pidx_hbm, tp_hbm, out_hbm, ib0, ib1, ib2, ib3, rows0, rows1,
          si0, si1, si2, si3, sg0, sg1, sw0, sw1):
        ib = (ib0, ib1, ib2, ib3)
        sem_i = (si0, si1, si2, si3)
        rows = (rows0, rows1)
        sem_g = (sg0, sg1)
        sem_w = (sw0, sw1)
        wid = lax.axis_index("s") * NC + lax.axis_index("c")
        base = wid * P_PER_W

        def start_iload(pair, v):
            pltpu.async_copy(
                pidx_hbm.at[pl.ds(base + pair * 2 * JP, 2 * JP)],
                ib[v], sem_i[v])

        def wait_iload(v):
            pltpu.make_async_copy(pidx_hbm.at[pl.ds(0, 2 * JP)], ib[v],
                                  sem_i[v]).wait()

        def start_gather(c, p, v, pos):
            pltpu.async_copy(
                tp_hbm.at[ib[v].at[pl.ds(pos * JP, JP)]], rows[p], sem_g[p])

        def wait_gather(p):
            pltpu.make_async_copy(out_hbm.at[pl.ds(0, JP)], rows[p],
                                  sem_g[p]).wait()

        def start_write(c, p):
            pltpu.async_copy(rows[p], out_hbm.at[pl.ds(base + c * JP, JP)],
                             sem_w[p])

        def wait_write(p):
            pltpu.make_async_copy(rows[p], out_hbm.at[pl.ds(0, JP)],
                                  sem_w[p]).wait()

        pltpu.sync_copy(pidx_hbm.at[pl.ds(base, 2 * JP)], ib0)
        start_iload(1, 1)
        start_gather(0, 0, 0, 0)

        def body(qi, carry):
            for u in range(4):
                i = 4 * qi + u
                c0 = 2 * i

                @pl.when(i < N_PAIR - 2)
                def _():
                    start_iload(i + 2, (u + 2) % 4)

                wait_gather(0)
                start_write(c0, 0)

                @pl.when(i > 0)
                def _():
                    wait_write(1)
                start_gather(c0 + 1, 1, u, 1)

                wait_gather(1)
                start_write(c0 + 1, 1)

                @pl.when(i < N_PAIR - 1)
                def _():
                    wait_write(0)
                    wait_iload((u + 1) % 4)
                    start_gather(c0 + 2, 0, (u + 1) % 4, 0)
            return carry

        lax.fori_loop(0, N_PAIR // 4, body, 0)
        wait_write(0)
        wait_write(1)

    return k(pidx, tp)


def kernel(left_idx, up_idx, table):
    n_pad = -table.shape[0] % 8
    tp = (jnp.pad(table, ((0, n_pad), (0, 0)))
          .reshape(-1, 8, 3, 128).transpose(0, 2, 1, 3).reshape(-1, 128))
    li = left_idx.reshape(B, 25, 8)
    ui = up_idx.reshape(B, 25, 8)

    def piece_base(r):
        return 24 * (r >> 3) + (r & 7)

    cb = jnp.arange(6, dtype=jnp.int32).reshape(1, 1, 6, 1)
    pidx = (jnp.where(cb < 3, piece_base(li)[:, :, None, :],
                      piece_base(ui)[:, :, None, :])
            + 8 * (cb % 3)).astype(jnp.int32).reshape(-1)

    out = _sc_piece_gather(pidx, tp)
    return (out.reshape(B, 25, 6, 8, 128)
               .transpose(0, 1, 3, 2, 4)
               .reshape(B, L, 6 * 128))
